# baseline (speedup 1.0000x reference)
import functools
import math

import jax
import jax.numpy as jnp
from jax import lax
from jax.experimental import pallas as pl
from jax.experimental.pallas import tpu as pltpu
from jax.experimental.pallas import tpu_sc as plsc

N = 10000
E = 160000
IN_DIM = 256
OUT_DIM = 256
HID = 32
INV_SQRT_D = 1.0 / math.sqrt(OUT_DIM)

NC = 2
NS = 16
NW = NC * NS

NBUF = 4
CEDGE = 128
CH_PER_W = 40
QPP = 20
CQ0 = 7
EP = NW * CH_PER_W * CEDGE
PADE = EP - E
NROWS = EP // CEDGE
NPAD = 10112
RPS = NPAD // NS
SRC_W = 80
DST_W = 48
MSG_W = 48

TCB = 1000


def _tables_body(obs_ref, w1_ref, b1_ref, par_ref, src_ref, dst_ref):
    h = jnp.tanh(
        jnp.dot(obs_ref[...], w1_ref[...], preferred_element_type=jnp.float32)
        + b1_ref[...])
    hk = h[:, 0:HID]
    hq = h[:, HID:2 * HID]
    hv = h[:, 2 * HID:3 * HID]
    par = par_ref[...]
    m = par[0:HID, :]
    u = par[HID:HID + 1, :]
    vvec = par[HID + 1:HID + 2, :]
    c = par[HID + 2, 0]
    ksc = jnp.dot(hk, m, preferred_element_type=jnp.float32)
    sbias = jnp.sum(hk * u, axis=1, keepdims=True) + c
    vq = jnp.sum(hq * vvec, axis=1, keepdims=True)
    one = jnp.ones((TCB, 1), jnp.float32)
    zpad = jnp.zeros((TCB, 14), jnp.float32)
    src_ref[...] = jnp.concatenate([ksc, sbias, one, zpad, hv], axis=1)
    dst_ref[...] = jnp.concatenate([hq, one, vq, zpad], axis=1)


_tc_tables = pl.pallas_call(
    _tables_body,
    grid=(N // TCB,),
    in_specs=[
        pl.BlockSpec((TCB, IN_DIM), lambda i: (i, 0)),
        pl.BlockSpec((IN_DIM, 3 * HID), lambda i: (0, 0)),
        pl.BlockSpec((1, 3 * HID), lambda i: (0, 0)),
        pl.BlockSpec((40, HID), lambda i: (0, 0)),
    ],
    out_specs=[
        pl.BlockSpec((TCB, SRC_W), lambda i: (i, 0)),
        pl.BlockSpec((TCB, DST_W), lambda i: (i, 0)),
    ],
    out_shape=[
        jax.ShapeDtypeStruct((N, SRC_W), jnp.float32),
        jax.ShapeDtypeStruct((N, DST_W), jnp.float32),
    ],
)


def _sc_body(srct_hbm, dstt_hbm, srci_hbm, dsti_hbm,
             part_out, alpha_out,
             srci_v, dsti_v,
             srow0, srow1, srow2, srow3,
             drow0, drow1, drow2, drow3,
             msg_v, alv_v, tr_v,
             acc_sh,
             semg0, semg1, semg2, semg3):
    cid = lax.axis_index("c")
    sid = lax.axis_index("s")
    wid = sid * NC + cid
    lane = lax.broadcasted_iota(jnp.int32, (16,), 0)
    zero16 = jnp.zeros((16,), jnp.float32)
    srow = (srow0, srow1, srow2, srow3)
    drow = (drow0, drow1, drow2, drow3)
    semg = (semg0, semg1, semg2, semg3)

    def _zrow(r, carry):
        for i in range(NBUF):
            msg_v[i, r, pl.ds(0, 16)] = zero16
            msg_v[i, r, pl.ds(16, 16)] = zero16
            msg_v[i, r, pl.ds(32, 16)] = zero16
        return carry

    lax.fori_loop(0, CEDGE, _zrow, 0)
    zbase = sid * RPS
    for t in range(RPS // CEDGE):
        pltpu.sync_copy(msg_v.at[0],
                        acc_sh.at[pl.ds(zbase + t * CEDGE, CEDGE)])
    rem = RPS % CEDGE
    if rem:
        pltpu.sync_copy(msg_v.at[0, pl.ds(0, rem)],
                        acc_sh.at[pl.ds(zbase + (RPS // CEDGE) * CEDGE, rem)])
    plsc.subcore_barrier()

    onehot0 = jnp.where(lane == 0, 1.0, 0.0).astype(jnp.float32)

    def _compute(b):
        def _group(g, carry2):
            base_e = g * 16
            for j in range(16):
                e = base_e + j
                p = None
                for v in range(3):
                    sv = srow[b][e, pl.ds(v * 16, 16)]
                    dv = drow[b][e, pl.ds(v * 16, 16)]
                    pv = sv * dv
                    p = pv if p is None else p + pv
                tr_v[j, pl.ds(0, 16)] = p
            cols = []
            for jj in range(16):
                cj = jnp.full((16,), jj, jnp.int32)
                cols.append(plsc.load_gather(tr_v, [lane, cj]))
            while len(cols) > 1:
                cols = [a + c for a, c in zip(cols[::2], cols[1::2])]
            av = 1.0 / (1.0 + jnp.exp(-cols[0] * INV_SQRT_D))
            alv_v[b, g] = av
            for j in range(16):
                e = base_e + j
                aj = jnp.broadcast_to(av[j], (16,))
                msg_v[b, e, pl.ds(0, 16)] = aj * srow[b][e, pl.ds(48, 16)]
                msg_v[b, e, pl.ds(16, 16)] = aj * srow[b][e, pl.ds(64, 16)]
                msg_v[b, e, pl.ds(32, 16)] = aj * onehot0
            return carry2

        lax.fori_loop(0, CEDGE // 16, _group, 0)

    nq = lax.select(cid == 0, jnp.int32(CQ0), jnp.int32(QPP - CQ0))
    wkbase = sid * (QPP * NBUF) + cid * (CQ0 * NBUF)

    def _wait_gathers(slot):
        for i in range(NBUF):
            pltpu.make_async_copy(
                srct_hbm.at[srci_v.at[slot * NBUF + i]],
                srow[i], semg[i]).wait()
            pltpu.make_async_copy(
                dstt_hbm.at[dsti_v.at[slot * NBUF + i]],
                drow[i], semg[i]).wait()

    def _load_idx_and_start(tnext, slot):
        nbase = wkbase + tnext * NBUF
        pltpu.sync_copy(srci_hbm.at[pl.ds(nbase, NBUF)],
                        srci_v.at[pl.ds(slot * NBUF, NBUF)])
        pltpu.sync_copy(dsti_hbm.at[pl.ds(nbase, NBUF)],
                        dsti_v.at[pl.ds(slot * NBUF, NBUF)])
        for i in range(NBUF):
            pltpu.async_copy(srct_hbm.at[srci_v.at[slot * NBUF + i]],
                             srow[i], semg[i])
            pltpu.async_copy(dstt_hbm.at[dsti_v.at[slot * NBUF + i]],
                             drow[i], semg[i])

    def _writes(t, slot):
        for i in range(NBUF):
            pltpu.sync_copy(msg_v.at[i],
                            acc_sh.at[dsti_v.at[slot * NBUF + i]], add=True)
        pltpu.sync_copy(alv_v,
                        alpha_out.at[pl.ds(wkbase + t * NBUF, NBUF)])

    _load_idx_and_start(0, 0)

    def _quad(t, carry):
        q = lax.rem(t, 2)
        _wait_gathers(q)
        for i in range(NBUF):
            _compute(i)
        _load_idx_and_start(t + 1, 1 - q)
        _writes(t, q)
        return carry

    lax.fori_loop(0, nq - 1, _quad, 0)
    ql = lax.rem(nq - 1, 2)
    _wait_gathers(ql)
    for i in range(NBUF):
        _compute(i)
    _writes(nq - 1, ql)
    plsc.subcore_barrier()

    pltpu.sync_copy(acc_sh.at[pl.ds(sid * RPS, RPS)],
                    part_out.at[pl.ds(cid * NPAD + sid * RPS, RPS)])


_sc_edge = functools.partial(
    pl.kernel,
    out_type=(
        jax.ShapeDtypeStruct((NC * NPAD, MSG_W), jnp.float32),
        jax.ShapeDtypeStruct((NROWS, CEDGE // 16, 16), jnp.float32),
    ),
    mesh=plsc.VectorSubcoreMesh(
        core_axis_name="c", subcore_axis_name="s",
        num_cores=NC, num_subcores=NS),
    compiler_params=pltpu.CompilerParams(
        needs_layout_passes=False, use_tc_tiling_on_sc=False),
    scratch_types=(
        [pltpu.VMEM((2 * NBUF, CEDGE), jnp.int32)] * 2
        + [pltpu.VMEM((CEDGE, SRC_W), jnp.float32)] * NBUF
        + [pltpu.VMEM((CEDGE, DST_W), jnp.float32)] * NBUF
        + [pltpu.VMEM((NBUF, CEDGE, MSG_W), jnp.float32)]
        + [pltpu.VMEM((NBUF, CEDGE // 16, 16), jnp.float32)]
        + [pltpu.VMEM((16, 17), jnp.float32)]
        + [pltpu.VMEM_SHARED((NPAD, MSG_W), jnp.float32)]
        + [pltpu.SemaphoreType.DMA] * NBUF
    ),
)(_sc_body)


def _expand_body(pa_ref, pb_ref, w2_ref, b2_ref, out_ref):
    s = pa_ref[...] + pb_ref[...]
    out_ref[...] = (
        jnp.dot(s[:, 0:HID], w2_ref[...], preferred_element_type=jnp.float32)
        + s[:, HID:HID + 1] * b2_ref[...])


_tc_expand = pl.pallas_call(
    _expand_body,
    grid=(N // TCB,),
    in_specs=[
        pl.BlockSpec((TCB, MSG_W), lambda i: (i, 0)),
        pl.BlockSpec((TCB, MSG_W), lambda i: (i, 0)),
        pl.BlockSpec((HID, OUT_DIM), lambda i: (0, 0)),
        pl.BlockSpec((1, OUT_DIM), lambda i: (0, 0)),
    ],
    out_specs=pl.BlockSpec((TCB, OUT_DIM), lambda i: (i, 0)),
    out_shape=jax.ShapeDtypeStruct((N, OUT_DIM), jnp.float32),
)


def kernel(observations, edge_index, Wk1, bk1, Wk2, bk2,
           Wq1, bq1, Wq2, bq2, Wv1, bv1, Wv2, bv2):
    f32 = jnp.float32
    w1 = jnp.concatenate([Wk1, Wq1, Wv1], axis=0).T.astype(f32)
    b1 = jnp.concatenate([bk1, bq1, bv1]).reshape(1, 3 * HID)
    m = Wk2.T @ Wq2
    u = Wk2.T @ bq2
    vvec = Wq2.T @ bk2
    c = jnp.dot(bk2, bq2)
    par = jnp.zeros((40, HID), f32)
    par = par.at[0:HID].set(m).at[HID].set(u).at[HID + 1].set(vvec)
    par = par.at[HID + 2, 0].set(c)

    src_t, dst_t = _tc_tables(observations, w1, b1, par)

    src = edge_index[0]
    dst = edge_index[1]
    srci = jnp.concatenate(
        [src, jnp.zeros((PADE,), jnp.int32)]).reshape(NROWS, CEDGE)
    dsti = jnp.concatenate(
        [dst, jnp.full((PADE,), N, jnp.int32)]).reshape(NROWS, CEDGE)

    part, alpha3 = _sc_edge(src_t, dst_t, srci, dsti)

    obs_proc = _tc_expand(part[:N], part[NPAD:NPAD + N],
                          Wv2.T.astype(f32), bv2.reshape(1, OUT_DIM))
    alpha = alpha3.reshape(EP)[:E].reshape(E, 1)
    return obs_proc, alpha


# speedup vs baseline: 1.2485x; 1.2485x over previous
import functools
import math

import jax
import jax.numpy as jnp
from jax import lax
from jax.experimental import pallas as pl
from jax.experimental.pallas import tpu as pltpu
from jax.experimental.pallas import tpu_sc as plsc

N = 10000
E = 160000
IN_DIM = 256
OUT_DIM = 256
HID = 32
INV_SQRT_D = 1.0 / math.sqrt(OUT_DIM)

NC = 2
NS = 16
NW = NC * NS

NBUF = 4
CEDGE = 128
CH_PER_W = 40
QPP = 20
CQ0 = 13
EP = NW * CH_PER_W * CEDGE
PADE = EP - E
NROWS = EP // CEDGE
NPAD = 10112
RPS = NPAD // NS
SRC_W = 80
DST_W = 48
MSG_W = 48

TCB = 1000


def _tables_body(obs_ref, w1_ref, b1_ref, par_ref, src_ref, dst_ref):
    h = jnp.tanh(
        jnp.dot(obs_ref[...], w1_ref[...], preferred_element_type=jnp.float32)
        + b1_ref[...])
    hk = h[:, 0:HID]
    hq = h[:, HID:2 * HID]
    hv = h[:, 2 * HID:3 * HID]
    par = par_ref[...]
    m = par[0:HID, :]
    u = par[HID:HID + 1, :]
    vvec = par[HID + 1:HID + 2, :]
    c = par[HID + 2, 0]
    ksc = jnp.dot(hk, m, preferred_element_type=jnp.float32)
    sbias = jnp.sum(hk * u, axis=1, keepdims=True) + c
    vq = jnp.sum(hq * vvec, axis=1, keepdims=True)
    one = jnp.ones((TCB, 1), jnp.float32)
    zpad = jnp.zeros((TCB, 14), jnp.float32)
    src_ref[...] = jnp.concatenate([ksc, sbias, one, zpad, hv], axis=1)
    dst_ref[...] = jnp.concatenate([hq, one, vq, zpad], axis=1)


_tc_tables = pl.pallas_call(
    _tables_body,
    grid=(N // TCB,),
    in_specs=[
        pl.BlockSpec((TCB, IN_DIM), lambda i: (i, 0)),
        pl.BlockSpec((IN_DIM, 3 * HID), lambda i: (0, 0)),
        pl.BlockSpec((1, 3 * HID), lambda i: (0, 0)),
        pl.BlockSpec((40, HID), lambda i: (0, 0)),
    ],
    out_specs=[
        pl.BlockSpec((TCB, SRC_W), lambda i: (i, 0)),
        pl.BlockSpec((TCB, DST_W), lambda i: (i, 0)),
    ],
    out_shape=[
        jax.ShapeDtypeStruct((N, SRC_W), jnp.float32),
        jax.ShapeDtypeStruct((N, DST_W), jnp.float32),
    ],
)


def _sc_body(srct_hbm, dstt_hbm, srci_hbm, dsti_hbm,
             part_out, alpha_out,
             srci_v, dsti_v,
             srow0, srow1, srow2, srow3,
             drow0, drow1, drow2, drow3,
             msg_v, alv_v, tr_v,
             acc_sh,
             semg0, semg1, semg2, semg3):
    cid = lax.axis_index("c")
    sid = lax.axis_index("s")
    wid = sid * NC + cid
    lane = lax.broadcasted_iota(jnp.int32, (16,), 0)
    zero16 = jnp.zeros((16,), jnp.float32)
    srow = (srow0, srow1, srow2, srow3)
    drow = (drow0, drow1, drow2, drow3)
    semg = (semg0, semg1, semg2, semg3)

    def _zrow(r, carry):
        for i in range(NBUF):
            msg_v[i, r, pl.ds(0, 16)] = zero16
            msg_v[i, r, pl.ds(16, 16)] = zero16
            msg_v[i, r, pl.ds(32, 16)] = zero16
        return carry

    lax.fori_loop(0, CEDGE, _zrow, 0)
    zbase = sid * RPS
    for t in range(RPS // CEDGE):
        pltpu.sync_copy(msg_v.at[0],
                        acc_sh.at[pl.ds(zbase + t * CEDGE, CEDGE)])
    rem = RPS % CEDGE
    if rem:
        pltpu.sync_copy(msg_v.at[0, pl.ds(0, rem)],
                        acc_sh.at[pl.ds(zbase + (RPS // CEDGE) * CEDGE, rem)])
    plsc.subcore_barrier()

    onehot0 = jnp.where(lane == 0, 1.0, 0.0).astype(jnp.float32)

    def _compute(b):
        def _group(g, carry2):
            base_e = g * 16
            for j in range(16):
                e = base_e + j
                p = None
                for v in range(3):
                    sv = srow[b][e, pl.ds(v * 16, 16)]
                    dv = drow[b][e, pl.ds(v * 16, 16)]
                    pv = sv * dv
                    p = pv if p is None else p + pv
                tr_v[j, pl.ds(0, 16)] = p
            cols = []
            for jj in range(16):
                cj = jnp.full((16,), jj, jnp.int32)
                cols.append(plsc.load_gather(tr_v, [lane, cj]))
            while len(cols) > 1:
                cols = [a + c for a, c in zip(cols[::2], cols[1::2])]
            av = 1.0 / (1.0 + jnp.exp(-cols[0] * INV_SQRT_D))
            alv_v[b, g] = av
            for j in range(16):
                e = base_e + j
                aj = jnp.broadcast_to(av[j], (16,))
                msg_v[b, e, pl.ds(0, 16)] = aj * srow[b][e, pl.ds(48, 16)]
                msg_v[b, e, pl.ds(16, 16)] = aj * srow[b][e, pl.ds(64, 16)]
                msg_v[b, e, pl.ds(32, 16)] = aj * onehot0
            return carry2

        lax.fori_loop(0, CEDGE // 16, _group, 0)

    nq = lax.select(cid == 0, jnp.int32(CQ0), jnp.int32(QPP - CQ0))
    wkbase = sid * (QPP * NBUF) + cid * (CQ0 * NBUF)

    def _wait_gathers(slot):
        for i in range(NBUF):
            pltpu.make_async_copy(
                srct_hbm.at[srci_v.at[slot * NBUF + i]],
                srow[i], semg[i]).wait()
            pltpu.make_async_copy(
                dstt_hbm.at[dsti_v.at[slot * NBUF + i]],
                drow[i], semg[i]).wait()

    def _load_idx_and_start(tnext, slot):
        nbase = wkbase + tnext * NBUF
        pltpu.sync_copy(srci_hbm.at[pl.ds(nbase, NBUF)],
                        srci_v.at[pl.ds(slot * NBUF, NBUF)])
        pltpu.sync_copy(dsti_hbm.at[pl.ds(nbase, NBUF)],
                        dsti_v.at[pl.ds(slot * NBUF, NBUF)])
        for i in range(NBUF):
            pltpu.async_copy(srct_hbm.at[srci_v.at[slot * NBUF + i]],
                             srow[i], semg[i])
            pltpu.async_copy(dstt_hbm.at[dsti_v.at[slot * NBUF + i]],
                             drow[i], semg[i])

    def _writes(t, slot):
        for i in range(NBUF):
            pltpu.sync_copy(msg_v.at[i],
                            acc_sh.at[dsti_v.at[slot * NBUF + i]], add=True)
        pltpu.sync_copy(alv_v,
                        alpha_out.at[pl.ds(wkbase + t * NBUF, NBUF)])

    _load_idx_and_start(0, 0)

    def _quad(t, carry):
        q = lax.rem(t, 2)
        _wait_gathers(q)
        for i in range(NBUF):
            _compute(i)
        _load_idx_and_start(t + 1, 1 - q)
        _writes(t, q)
        return carry

    lax.fori_loop(0, nq - 1, _quad, 0)
    ql = lax.rem(nq - 1, 2)
    _wait_gathers(ql)
    for i in range(NBUF):
        _compute(i)
    _writes(nq - 1, ql)
    plsc.subcore_barrier()

    pltpu.sync_copy(acc_sh.at[pl.ds(sid * RPS, RPS)],
                    part_out.at[pl.ds(cid * NPAD + sid * RPS, RPS)])


_sc_edge = functools.partial(
    pl.kernel,
    out_type=(
        jax.ShapeDtypeStruct((NC * NPAD, MSG_W), jnp.float32),
        jax.ShapeDtypeStruct((NROWS, CEDGE // 16, 16), jnp.float32),
    ),
    mesh=plsc.VectorSubcoreMesh(
        core_axis_name="c", subcore_axis_name="s",
        num_cores=NC, num_subcores=NS),
    compiler_params=pltpu.CompilerParams(
        needs_layout_passes=False, use_tc_tiling_on_sc=False),
    scratch_types=(
        [pltpu.VMEM((2 * NBUF, CEDGE), jnp.int32)] * 2
        + [pltpu.VMEM((CEDGE, SRC_W), jnp.float32)] * NBUF
        + [pltpu.VMEM((CEDGE, DST_W), jnp.float32)] * NBUF
        + [pltpu.VMEM((NBUF, CEDGE, MSG_W), jnp.float32)]
        + [pltpu.VMEM((NBUF, CEDGE // 16, 16), jnp.float32)]
        + [pltpu.VMEM((16, 17), jnp.float32)]
        + [pltpu.VMEM_SHARED((NPAD, MSG_W), jnp.float32)]
        + [pltpu.SemaphoreType.DMA] * NBUF
    ),
)(_sc_body)


def _expand_body(pa_ref, pb_ref, w2_ref, b2_ref, out_ref):
    s = pa_ref[...] + pb_ref[...]
    out_ref[...] = (
        jnp.dot(s[:, 0:HID], w2_ref[...], preferred_element_type=jnp.float32)
        + s[:, HID:HID + 1] * b2_ref[...])


_tc_expand = pl.pallas_call(
    _expand_body,
    grid=(N // TCB,),
    in_specs=[
        pl.BlockSpec((TCB, MSG_W), lambda i: (i, 0)),
        pl.BlockSpec((TCB, MSG_W), lambda i: (i, 0)),
        pl.BlockSpec((HID, OUT_DIM), lambda i: (0, 0)),
        pl.BlockSpec((1, OUT_DIM), lambda i: (0, 0)),
    ],
    out_specs=pl.BlockSpec((TCB, OUT_DIM), lambda i: (i, 0)),
    out_shape=jax.ShapeDtypeStruct((N, OUT_DIM), jnp.float32),
)


def kernel(observations, edge_index, Wk1, bk1, Wk2, bk2,
           Wq1, bq1, Wq2, bq2, Wv1, bv1, Wv2, bv2):
    f32 = jnp.float32
    w1 = jnp.concatenate([Wk1, Wq1, Wv1], axis=0).T.astype(f32)
    b1 = jnp.concatenate([bk1, bq1, bv1]).reshape(1, 3 * HID)
    m = Wk2.T @ Wq2
    u = Wk2.T @ bq2
    vvec = Wq2.T @ bk2
    c = jnp.dot(bk2, bq2)
    par = jnp.zeros((40, HID), f32)
    par = par.at[0:HID].set(m).at[HID].set(u).at[HID + 1].set(vvec)
    par = par.at[HID + 2, 0].set(c)

    src_t, dst_t = _tc_tables(observations, w1, b1, par)

    src = edge_index[0]
    dst = edge_index[1]
    srci = jnp.concatenate(
        [src, jnp.zeros((PADE,), jnp.int32)]).reshape(NROWS, CEDGE)
    dsti = jnp.concatenate(
        [dst, jnp.full((PADE,), N, jnp.int32)]).reshape(NROWS, CEDGE)

    part, alpha3 = _sc_edge(src_t, dst_t, srci, dsti)

    obs_proc = _tc_expand(part[:N], part[NPAD:NPAD + N],
                          Wv2.T.astype(f32), bv2.reshape(1, OUT_DIM))
    alpha = alpha3.reshape(EP)[:E].reshape(E, 1)
    return obs_proc, alpha
